# baseline (speedup 1.0000x reference)
import functools

import jax
import jax.numpy as jnp
from jax import lax
from jax.experimental import pallas as pl
from jax.experimental.pallas import tpu as pltpu
from jax.experimental.pallas import tpu_sc as plsc

_SEQ_LEN = 8192
_N_DIMS = 128
_NUM_CORES = 2
_NUM_SUBCORES = 16
_NUM_WORKERS = _NUM_CORES * _NUM_SUBCORES
_ROWS_PER_W = _SEQ_LEN // _NUM_WORKERS

_mesh = plsc.VectorSubcoreMesh(core_axis_name="c", subcore_axis_name="s")


_ROWS_W0 = 288
_ROWS_W1 = 224


@functools.partial(
    pl.kernel,
    mesh=_mesh,
    out_type=jax.ShapeDtypeStruct((1, _SEQ_LEN, _N_DIMS), jnp.float32),
    scratch_types=[
        pltpu.VMEM((_ROWS_W0, _N_DIMS), jnp.float32),
    ],
)
def _position_lookup(table_hbm, out_hbm, buf_v):
    c = lax.axis_index("c")
    s = lax.axis_index("s")

    @pl.when(c == 0)
    def _():
        base = s * _ROWS_W0
        pltpu.sync_copy(table_hbm.at[pl.ds(base, _ROWS_W0)],
                        buf_v.at[pl.ds(0, _ROWS_W0)])
        pltpu.sync_copy(buf_v.at[pl.ds(0, _ROWS_W0)],
                        out_hbm.at[0, pl.ds(base, _ROWS_W0)])

    @pl.when(c == 1)
    def _():
        base = _NUM_SUBCORES * _ROWS_W0 + s * _ROWS_W1
        pltpu.sync_copy(table_hbm.at[pl.ds(base, _ROWS_W1)],
                        buf_v.at[pl.ds(0, _ROWS_W1)])
        pltpu.sync_copy(buf_v.at[pl.ds(0, _ROWS_W1)],
                        out_hbm.at[0, pl.ds(base, _ROWS_W1)])


def kernel(position_embed):
    return _position_lookup(position_embed)


# speedup vs baseline: 1.0233x; 1.0233x over previous
import functools

import jax
import jax.numpy as jnp
from jax import lax
from jax.experimental import pallas as pl
from jax.experimental.pallas import tpu as pltpu
from jax.experimental.pallas import tpu_sc as plsc

_SEQ_LEN = 8192
_N_DIMS = 128
_NUM_CORES = 2
_NUM_SUBCORES = 16
_NUM_WORKERS = _NUM_CORES * _NUM_SUBCORES
_ROWS_PER_W = _SEQ_LEN // _NUM_WORKERS

_mesh = plsc.VectorSubcoreMesh(core_axis_name="c", subcore_axis_name="s")


_ROWS_W0 = 224
_ROWS_W1 = 288


@functools.partial(
    pl.kernel,
    mesh=_mesh,
    out_type=jax.ShapeDtypeStruct((1, _SEQ_LEN, _N_DIMS), jnp.float32),
    scratch_types=[
        pltpu.VMEM((_ROWS_W1, _N_DIMS), jnp.float32),
    ],
)
def _position_lookup(table_hbm, out_hbm, buf_v):
    c = lax.axis_index("c")
    s = lax.axis_index("s")

    @pl.when(c == 0)
    def _():
        base = s * _ROWS_W0
        pltpu.sync_copy(table_hbm.at[pl.ds(base, _ROWS_W0)],
                        buf_v.at[pl.ds(0, _ROWS_W0)])
        pltpu.sync_copy(buf_v.at[pl.ds(0, _ROWS_W0)],
                        out_hbm.at[0, pl.ds(base, _ROWS_W0)])

    @pl.when(c == 1)
    def _():
        base = _NUM_SUBCORES * _ROWS_W0 + s * _ROWS_W1
        pltpu.sync_copy(table_hbm.at[pl.ds(base, _ROWS_W1)],
                        buf_v.at[pl.ds(0, _ROWS_W1)])
        pltpu.sync_copy(buf_v.at[pl.ds(0, _ROWS_W1)],
                        out_hbm.at[0, pl.ds(base, _ROWS_W1)])


def kernel(position_embed):
    return _position_lookup(position_embed)
